# Initial kernel scaffold; baseline (speedup 1.0000x reference)
import jax
import jax.numpy as jnp
from jax.experimental import pallas as pl


def kernel(h_channel):
    raise NotImplementedError("write your pallas kernel here")


# speedup vs baseline: 1.8605x; 1.8605x over previous
import functools

import jax
import jax.numpy as jnp
from jax import lax
from jax.experimental import pallas as pl
from jax.experimental.pallas import tpu as pltpu
from jax.experimental.pallas import tpu_sc as plsc

B, NRX, NRA, NTX, NTA = 4, 4, 4, 1, 64
NOFDM, NSC = 14, 512
D = NOFDM * NSC
NUM_BEAMS = 16
K = NUM_BEAMS // NRX
ROWS_IN = B * NRX * NRA * NTA
ROWS_OUT = B * NRX * NRA * NUM_BEAMS
SCALE = 0.25

NC, NS, L = 2, 16, 16
NW = NC * NS
ROWS_PER_W = ROWS_OUT // NW
CHUNK = 16
NCHUNK = ROWS_PER_W // CHUNK


def _power_topk_body(h_ref, idx_ref, acc_ref):
    a = pl.program_id(2)
    x = h_ref[0, 0, 0]
    part = jnp.sum(x * x, axis=1)

    @pl.when(a == 0)
    def _():
        acc_ref[0, :] = part

    @pl.when(a > 0)
    def _():
        acc_ref[0, :] = acc_ref[0, :] + part

    @pl.when(a == NRA - 1)
    def _():
        p2 = acc_ref[...]
        iota = lax.broadcasted_iota(jnp.int32, (1, NTA), 1)
        kiota = lax.broadcasted_iota(jnp.int32, (1, K), 1)
        idx_out = jnp.zeros((1, K), jnp.int32)
        for k in range(K):
            mx = jnp.max(p2)
            j = jnp.min(jnp.where(p2 == mx, iota, NTA))
            idx_out = jnp.where(kiota == k, j, idx_out)
            p2 = jnp.where(iota == j, -1.0, p2)
        idx_ref[0, 0] = idx_out


def _power_topk(h5):
    return pl.pallas_call(
        _power_topk_body,
        grid=(B, NRX, NRA),
        in_specs=[pl.BlockSpec((1, 1, 1, NTA, D), lambda b, r, a: (b, r, a, 0, 0))],
        out_specs=pl.BlockSpec((1, 1, 1, K), lambda b, r, a: (b, r, 0, 0)),
        out_shape=jax.ShapeDtypeStruct((B, NRX, 1, K), jnp.int32),
        scratch_shapes=[pltpu.VMEM((1, NTA), jnp.float32)],
    )(h5)


@functools.cache
def _make_sc_gather():
    def body(table_hbm, idx_hbm, out_hbm, idx_v, trow_v, rows_v, sem):
        cid = lax.axis_index("c")
        sid = lax.axis_index("s")
        wid = sid * NC + cid
        pltpu.sync_copy(idx_hbm, idx_v)
        for c in range(NCHUNK):
            grp = wid * NCHUNK + c
            b = grp // NUM_BEAMS
            beam = idx_v[pl.ds(b * NUM_BEAMS, L)]
            trow_v[...] = grp * NTA + beam
            pltpu.async_copy(table_hbm.at[trow_v], rows_v, sem).wait()

            def srow(rr, carry):
                def scol(t, carry2):
                    sl = pl.ds(t * L, L)
                    rows_v[rr, sl] = rows_v[rr, sl] * SCALE
                    return carry2
                return lax.fori_loop(0, D // L, scol, carry)

            lax.fori_loop(0, CHUNK, srow, 0)
            base = wid * ROWS_PER_W + c * CHUNK
            pltpu.sync_copy(rows_v, out_hbm.at[pl.ds(base, CHUNK)])

    return pl.kernel(
        body,
        out_type=jax.ShapeDtypeStruct((ROWS_OUT, D), jnp.float32),
        mesh=plsc.VectorSubcoreMesh(core_axis_name="c", subcore_axis_name="s"),
        scratch_types=[
            pltpu.VMEM((B * NUM_BEAMS,), jnp.int32),
            pltpu.VMEM((CHUNK,), jnp.int32),
            pltpu.VMEM((CHUNK, D), jnp.float32),
            pltpu.SemaphoreType.DMA,
        ],
    )


def kernel(h_channel):
    h5 = h_channel.reshape(B, NRX, NRA, NTA, D)
    idx = _power_topk(h5)
    idx_flat = idx.reshape(B * NUM_BEAMS)
    table = h_channel.reshape(ROWS_IN, D)
    out = _make_sc_gather()(table, idx_flat)
    return out.reshape(B, NRX, NRA, NUM_BEAMS, NOFDM, NSC)[:, :, :, None]


# speedup vs baseline: 2.5799x; 1.3867x over previous
import functools

import jax
import jax.numpy as jnp
from jax import lax
from jax.experimental import pallas as pl
from jax.experimental.pallas import tpu as pltpu
from jax.experimental.pallas import tpu_sc as plsc

B, NRX, NRA, NTX, NTA = 4, 4, 4, 1, 64
NOFDM, NSC = 14, 512
D = NOFDM * NSC
NUM_BEAMS = 16
K = NUM_BEAMS // NRX
ROWS_IN = B * NRX * NRA * NTA
ROWS_OUT = B * NRX * NRA * NUM_BEAMS
SCALE = 0.25

NC, NS, L = 2, 16, 16
NW = NC * NS
ROWS_PER_W = ROWS_OUT // NW
CHUNK = 16
NCHUNK = ROWS_PER_W // CHUNK


def _power_topk_body(h_ref, idx_ref, acc_ref):
    a = pl.program_id(2)
    x = h_ref[0, 0, 0, 0]
    part = jnp.sum(x * x, axis=(1, 2))

    @pl.when(a == 0)
    def _():
        acc_ref[0, :] = part

    @pl.when(a > 0)
    def _():
        acc_ref[0, :] = acc_ref[0, :] + part

    @pl.when(a == NRA - 1)
    def _():
        p2 = acc_ref[...]
        iota = lax.broadcasted_iota(jnp.int32, (1, NTA), 1)
        kiota = lax.broadcasted_iota(jnp.int32, (1, K), 1)
        idx_out = jnp.zeros((1, K), jnp.int32)
        for k in range(K):
            mx = jnp.max(p2)
            j = jnp.min(jnp.where(p2 == mx, iota, NTA))
            idx_out = jnp.where(kiota == k, j, idx_out)
            p2 = jnp.where(iota == j, -1.0, p2)
        idx_ref[0, 0] = idx_out


def _power_topk(h7):
    return pl.pallas_call(
        _power_topk_body,
        grid=(B, NRX, NRA),
        in_specs=[pl.BlockSpec((1, 1, 1, 1, NTA, NOFDM, NSC),
                               lambda b, r, a: (b, r, a, 0, 0, 0, 0))],
        out_specs=pl.BlockSpec((1, 1, 1, K), lambda b, r, a: (b, r, 0, 0)),
        out_shape=jax.ShapeDtypeStruct((B, NRX, 1, K), jnp.int32),
        scratch_shapes=[pltpu.VMEM((1, NTA), jnp.float32)],
    )(h7)


NBUF = 2


@functools.cache
def _make_sc_gather():
    def body(table_hbm, idx_hbm, out_hbm, idx_v, bufs, in_sems, out_sems):
        cid = lax.axis_index("c")
        sid = lax.axis_index("s")
        wid = sid * NC + cid
        pltpu.sync_copy(idx_hbm, idx_v)

        def issue(j):
            grp = wid * (ROWS_PER_W // NUM_BEAMS) + j // NUM_BEAMS
            k = j % NUM_BEAMS
            b = grp // NUM_BEAMS
            bvec = idx_v[pl.ds(b * NUM_BEAMS, L)]
            trow = grp * NTA + bvec[k]
            m = j % NBUF
            return pltpu.async_copy(table_hbm.at[trow], bufs.at[m], in_sems.at[m])

        def scale(m):
            def srow(o, carry):
                def scol(q, carry2):
                    sl = pl.ds(q * L, L)
                    bufs[m, o, sl] = bufs[m, o, sl] * SCALE
                    return carry2
                return lax.fori_loop(0, NSC // L, scol, carry)
            lax.fori_loop(0, NOFDM, srow, 0)

        in_cp = {0: issue(0)}
        out_cp = {}
        for j in range(ROWS_PER_W):
            m = j % NBUF
            if j + 1 < ROWS_PER_W:
                if j + 1 >= NBUF:
                    out_cp[(j + 1) % NBUF].wait()
                in_cp[j + 1] = issue(j + 1)
            in_cp[j].wait()
            scale(m)
            out_cp[m] = pltpu.async_copy(
                bufs.at[m], out_hbm.at[wid * ROWS_PER_W + j], out_sems.at[m])
        for m in range(min(NBUF, ROWS_PER_W)):
            out_cp[m].wait()

    return pl.kernel(
        body,
        out_type=jax.ShapeDtypeStruct((ROWS_OUT, NOFDM, NSC), jnp.float32),
        mesh=plsc.VectorSubcoreMesh(core_axis_name="c", subcore_axis_name="s"),
        scratch_types=[
            pltpu.VMEM((B * NUM_BEAMS,), jnp.int32),
            pltpu.VMEM((NBUF, NOFDM, NSC), jnp.float32),
            pltpu.SemaphoreType.DMA((NBUF,)),
            pltpu.SemaphoreType.DMA((NBUF,)),
        ],
    )


def kernel(h_channel):
    idx = _power_topk(h_channel)
    idx_flat = idx.reshape(B * NUM_BEAMS)
    table = h_channel.reshape(ROWS_IN, NOFDM, NSC)
    out = _make_sc_gather()(table, idx_flat)
    return out.reshape(B, NRX, NRA, NUM_BEAMS, NOFDM, NSC)[:, :, :, None]


# speedup vs baseline: 5.1572x; 1.9990x over previous
import functools

import jax
import jax.numpy as jnp
from jax import lax
from jax.experimental import pallas as pl
from jax.experimental.pallas import tpu as pltpu
from jax.experimental.pallas import tpu_sc as plsc

B, NRX, NRA, NTX, NTA = 4, 4, 4, 1, 64
NOFDM, NSC = 14, 512
NUM_BEAMS = 16
K = NUM_BEAMS // NRX
NGRP = B * NRX * NRA * NTX * NOFDM
ROWS_IN = NGRP * NTA
ROWS_OUT = NGRP * NUM_BEAMS
SCALE = 0.25

NC, NS, L = 2, 16, 16
NW = NC * NS
GRP_PER_W = NGRP // NW
GRP_PER_B = NRX * NRA * NOFDM
NBUF = 2


def _power_topk_body(h_ref, idx_ref, acc_ref):
    a = pl.program_id(2)
    x = h_ref[0, 0, 0, 0]
    part = jnp.sum(x * x, axis=(0, 2))

    @pl.when(a == 0)
    def _():
        acc_ref[0, :] = part

    @pl.when(a > 0)
    def _():
        acc_ref[0, :] = acc_ref[0, :] + part

    @pl.when(a == NRA - 1)
    def _():
        p2 = acc_ref[...]
        iota = lax.broadcasted_iota(jnp.int32, (1, NTA), 1)
        kiota = lax.broadcasted_iota(jnp.int32, (1, K), 1)
        idx_out = jnp.zeros((1, K), jnp.int32)
        for k in range(K):
            mx = jnp.max(p2)
            j = jnp.min(jnp.where(p2 == mx, iota, NTA))
            idx_out = jnp.where(kiota == k, j, idx_out)
            p2 = jnp.where(iota == j, -1.0, p2)
        idx_ref[0, 0] = idx_out


def _power_topk(hT):
    return pl.pallas_call(
        _power_topk_body,
        grid=(B, NRX, NRA),
        in_specs=[pl.BlockSpec((1, 1, 1, 1, NOFDM, NTA, NSC),
                               lambda b, r, a: (b, r, a, 0, 0, 0, 0))],
        out_specs=pl.BlockSpec((1, 1, 1, K), lambda b, r, a: (b, r, 0, 0)),
        out_shape=jax.ShapeDtypeStruct((B, NRX, 1, K), jnp.int32),
        scratch_shapes=[pltpu.VMEM((1, NTA), jnp.float32)],
    )(hT)


@functools.cache
def _make_sc_gather():
    def body(table_hbm, idx_hbm, out_hbm, idx_v, trow_v, bufs, in_sems, out_sems):
        cid = lax.axis_index("c")
        sid = lax.axis_index("s")
        wid = sid * NC + cid
        pltpu.sync_copy(idx_hbm, idx_v)

        def issue(t):
            grp = wid * GRP_PER_W + t
            b = grp // GRP_PER_B
            beams = idx_v[pl.ds(b * NUM_BEAMS, L)]
            m = t % NBUF
            trow_v[m, :] = grp * NTA + beams
            return pltpu.async_copy(
                table_hbm.at[trow_v.at[m]], bufs.at[m], in_sems.at[m])

        def scale(m):
            def srow(rr, carry):
                def scol(q, carry2):
                    sl = pl.ds(q * L, L)
                    bufs[m, rr, sl] = bufs[m, rr, sl] * SCALE
                    return carry2
                return lax.fori_loop(0, NSC // L, scol, carry)
            lax.fori_loop(0, NUM_BEAMS, srow, 0)

        in_cp = {0: issue(0)}
        out_cp = {}
        for t in range(GRP_PER_W):
            m = t % NBUF
            if t + 1 < GRP_PER_W:
                if t + 1 >= NBUF:
                    out_cp[(t + 1) % NBUF].wait()
                in_cp[t + 1] = issue(t + 1)
            in_cp[t].wait()
            scale(m)
            grp = wid * GRP_PER_W + t
            out_cp[m] = pltpu.async_copy(
                bufs.at[m], out_hbm.at[pl.ds(grp * NUM_BEAMS, NUM_BEAMS)],
                out_sems.at[m])
        for m in range(min(NBUF, GRP_PER_W)):
            out_cp[m].wait()

    return pl.kernel(
        body,
        out_type=jax.ShapeDtypeStruct((ROWS_OUT, NSC), jnp.float32),
        mesh=plsc.VectorSubcoreMesh(core_axis_name="c", subcore_axis_name="s"),
        scratch_types=[
            pltpu.VMEM((B * NUM_BEAMS,), jnp.int32),
            pltpu.VMEM((NBUF, NUM_BEAMS), jnp.int32),
            pltpu.VMEM((NBUF, NUM_BEAMS, NSC), jnp.float32),
            pltpu.SemaphoreType.DMA((NBUF,)),
            pltpu.SemaphoreType.DMA((NBUF,)),
        ],
    )


def kernel(h_channel):
    hT = jnp.transpose(h_channel, (0, 1, 2, 3, 5, 4, 6))
    idx = _power_topk(hT)
    idx_flat = idx.reshape(B * NUM_BEAMS)
    table = hT.reshape(ROWS_IN, NSC)
    outT = _make_sc_gather()(table, idx_flat)
    out = outT.reshape(B, NRX, NRA, NTX, NOFDM, NUM_BEAMS, NSC)
    return jnp.transpose(out, (0, 1, 2, 3, 5, 4, 6))


# speedup vs baseline: 10.5231x; 2.0404x over previous
import functools

import jax
import jax.numpy as jnp
from jax import lax
from jax.experimental import pallas as pl
from jax.experimental.pallas import tpu as pltpu
from jax.experimental.pallas import tpu_sc as plsc

B, NRX, NRA, NTX, NTA = 4, 4, 4, 1, 64
NOFDM, NSC = 14, 512
NUM_BEAMS = 16
K = NUM_BEAMS // NRX
NGRP = B * NRX * NRA * NTX * NOFDM
ROWS_IN = NGRP * NTA
ROWS_OUT = NGRP * NUM_BEAMS
SCALE = 0.25

NC, NS, L = 2, 16, 16
NW = NC * NS
GRP_PER_W = NGRP // NW
GRP_PER_B = NRX * NRA * NOFDM
NBUF = 3


def _power_topk_body(h_ref, idx_ref):
    x = h_ref[0, 0, :, 0]
    p = jnp.sum(x * x, axis=(0, 1, 3))
    p2 = p.reshape(1, NTA)
    iota = lax.broadcasted_iota(jnp.int32, (1, NTA), 1)
    kiota = lax.broadcasted_iota(jnp.int32, (1, K), 1)
    idx_out = jnp.zeros((1, K), jnp.int32)
    for k in range(K):
        mx = jnp.max(p2)
        j = jnp.min(jnp.where(p2 == mx, iota, NTA))
        idx_out = jnp.where(kiota == k, j, idx_out)
        p2 = jnp.where(iota == j, -1.0, p2)
    idx_ref[0, 0] = idx_out


def _power_topk(hT):
    return pl.pallas_call(
        _power_topk_body,
        grid=(B, NRX),
        in_specs=[pl.BlockSpec((1, 1, NRA, 1, NOFDM, NTA, NSC),
                               lambda b, r: (b, r, 0, 0, 0, 0, 0))],
        out_specs=pl.BlockSpec((1, 1, 1, K), lambda b, r: (b, r, 0, 0)),
        out_shape=jax.ShapeDtypeStruct((B, NRX, 1, K), jnp.int32),
    )(hT)


@functools.cache
def _make_sc_gather():
    def body(table_hbm, idx_hbm, out_hbm, idx_v, trow_v, bufs, in_sems, out_sems):
        cid = lax.axis_index("c")
        sid = lax.axis_index("s")
        wid = sid * NC + cid
        pltpu.sync_copy(idx_hbm, idx_v)

        def issue(t):
            grp = wid * GRP_PER_W + t
            b = grp // GRP_PER_B
            beams = idx_v[pl.ds(b * NUM_BEAMS, L)]
            m = t % NBUF
            trow_v[m, :] = grp * NTA + beams
            return pltpu.async_copy(
                table_hbm.at[trow_v.at[m]], bufs.at[m], in_sems.at[m])

        UNROLL = 8

        def scale(m):
            def srow(rr, carry):
                def scol(q, carry2):
                    for u in range(UNROLL):
                        sl = pl.ds((q * UNROLL + u) * L, L)
                        bufs[m, rr, sl] = bufs[m, rr, sl] * SCALE
                    return carry2
                return lax.fori_loop(0, NSC // (L * UNROLL), scol, carry)
            lax.fori_loop(0, NUM_BEAMS, srow, 0)

        in_cp = {0: issue(0)}
        out_cp = {}
        for t in range(GRP_PER_W):
            m = t % NBUF
            if t + 1 < GRP_PER_W:
                if t + 1 >= NBUF:
                    out_cp[(t + 1) % NBUF].wait()
                in_cp[t + 1] = issue(t + 1)
            in_cp[t].wait()
            scale(m)
            grp = wid * GRP_PER_W + t
            out_cp[m] = pltpu.async_copy(
                bufs.at[m], out_hbm.at[pl.ds(grp * NUM_BEAMS, NUM_BEAMS)],
                out_sems.at[m])
        for m in range(min(NBUF, GRP_PER_W)):
            out_cp[m].wait()

    return pl.kernel(
        body,
        out_type=jax.ShapeDtypeStruct((ROWS_OUT, NSC), jnp.float32),
        mesh=plsc.VectorSubcoreMesh(core_axis_name="c", subcore_axis_name="s"),
        scratch_types=[
            pltpu.VMEM((B * NUM_BEAMS,), jnp.int32),
            pltpu.VMEM((NBUF, NUM_BEAMS), jnp.int32),
            pltpu.VMEM((NBUF, NUM_BEAMS, NSC), jnp.float32),
            pltpu.SemaphoreType.DMA((NBUF,)),
            pltpu.SemaphoreType.DMA((NBUF,)),
        ],
    )


def kernel(h_channel):
    hT = jnp.transpose(h_channel, (0, 1, 2, 3, 5, 4, 6))
    idx = _power_topk(hT)
    idx_flat = idx.reshape(B * NUM_BEAMS)
    table = hT.reshape(ROWS_IN, NSC)
    outT = _make_sc_gather()(table, idx_flat)
    out = outT.reshape(B, NRX, NRA, NTX, NOFDM, NUM_BEAMS, NSC)
    return jnp.transpose(out, (0, 1, 2, 3, 5, 4, 6))
